# baseline (speedup 1.0000x reference)
import functools
import jax
import jax.numpy as jnp
from jax import lax
from jax.experimental import pallas as pl
from jax.experimental.pallas import tpu as pltpu
from jax.experimental.pallas import tpu_sc as plsc

_EPS = 1e-15
_B = 16384
_L = 20
_D = 64
_NC = 2
_NS = 16
_NW = _NC * _NS
_BPW = _B // _NW
_CH = 128
_CTX_CHUNKS = _BPW * _L // _CH
_TGT_CHUNKS = _BPW // _CH
_SC_B = _NS * _BPW


def _sc_body(tgt_idx_hbm, ctxp_hbm, ctxn_hbm, seg_hbm,
             tgt_tab_hbm, ctx_tab_hbm,
             outp_hbm, outn_hbm,
             tgt_idx_v, ctxp_v, ctxn_v, seg_v,
             rows_v, bufp_v, bufn_v,
             accp_sh, accn_sh, gsem):
    c = lax.axis_index("c")
    s = lax.axis_index("s")
    wid = c * _NS + s
    base = wid * _BPW
    sbase = s * _BPW

    pltpu.sync_copy(tgt_idx_hbm.at[wid], tgt_idx_v)
    pltpu.sync_copy(ctxp_hbm.at[wid], ctxp_v)
    pltpu.sync_copy(ctxn_hbm.at[wid], ctxn_v)
    pltpu.sync_copy(seg_hbm.at[s], seg_v)

    def _zero(r, carry):
        for cc in range(_D // 16):
            rows_v[r, pl.ds(cc * 16, 16)] = jnp.zeros((16,), jnp.float32)
        return carry
    lax.fori_loop(0, _CH, _zero, 0)
    for j in range(_TGT_CHUNKS):
        pltpu.sync_copy(rows_v, accp_sh.at[pl.ds(sbase + j * _CH, _CH)])
        pltpu.sync_copy(rows_v, accn_sh.at[pl.ds(sbase + j * _CH, _CH)])

    def _ctx(j, carry):
        pltpu.async_copy(ctx_tab_hbm.at[ctxp_v.at[j]], rows_v, gsem).wait()
        pltpu.sync_copy(rows_v, accp_sh.at[seg_v.at[j]], add=True)
        pltpu.async_copy(ctx_tab_hbm.at[ctxn_v.at[j]], rows_v, gsem).wait()
        pltpu.sync_copy(rows_v, accn_sh.at[seg_v.at[j]], add=True)
        return carry
    lax.fori_loop(0, _CTX_CHUNKS, _ctx, 0)

    for j in range(_TGT_CHUNKS):
        pltpu.async_copy(tgt_tab_hbm.at[tgt_idx_v.at[j]], rows_v, gsem).wait()
        pltpu.sync_copy(accp_sh.at[pl.ds(sbase + j * _CH, _CH)], bufp_v)
        pltpu.sync_copy(accn_sh.at[pl.ds(sbase + j * _CH, _CH)], bufn_v)

        def _ew(r, carry):
            for cc in range(_D // 16):
                sl = pl.ds(cc * 16, 16)
                t = rows_v[r, sl]
                bufp_v[r, sl] = t * bufp_v[r, sl] + _EPS
                bufn_v[r, sl] = 1.0 - (t * bufn_v[r, sl] + _EPS)
            return carry
        lax.fori_loop(0, _CH, _ew, 0)

        pltpu.sync_copy(bufp_v, outp_hbm.at[pl.ds(base + j * _CH, _CH)])
        pltpu.sync_copy(bufn_v, outn_hbm.at[pl.ds(base + j * _CH, _CH)])


_sc_scores = functools.partial(
    pl.kernel,
    out_type=(jax.ShapeDtypeStruct((_B, _D), jnp.float32),
              jax.ShapeDtypeStruct((_B, _D), jnp.float32)),
    mesh=plsc.VectorSubcoreMesh(core_axis_name="c", subcore_axis_name="s",
                                num_cores=_NC, num_subcores=_NS),
    scratch_types=[
        pltpu.VMEM((_TGT_CHUNKS, _CH), jnp.int32),
        pltpu.VMEM((_CTX_CHUNKS, _CH), jnp.int32),
        pltpu.VMEM((_CTX_CHUNKS, _CH), jnp.int32),
        pltpu.VMEM((_CTX_CHUNKS, _CH), jnp.int32),
        pltpu.VMEM((_CH, _D), jnp.float32),
        pltpu.VMEM((_CH, _D), jnp.float32),
        pltpu.VMEM((_CH, _D), jnp.float32),
        pltpu.VMEM_SHARED((_SC_B, _D), jnp.float32),
        pltpu.VMEM_SHARED((_SC_B, _D), jnp.float32),
        pltpu.SemaphoreType.DMA,
    ],
    compiler_params=pltpu.CompilerParams(use_tc_tiling_on_sc=False),
)(_sc_body)


def _loss_body(p_ref, n_ref, o_ref):
    xp = -p_ref[...]
    xn = -n_ref[...]
    sp = jnp.maximum(xp, 0.0) + jnp.log1p(jnp.exp(-jnp.abs(xp)))
    sn = jnp.maximum(xn, 0.0) + jnp.log1p(jnp.exp(-jnp.abs(xn)))
    o_ref[0, 0] = (jnp.sum(sp) + jnp.sum(sn)) * (1.0 / (_B * _D))


_loss = pl.pallas_call(
    _loss_body,
    out_shape=jax.ShapeDtypeStruct((1, 1), jnp.float32),
    out_specs=pl.BlockSpec(memory_space=pltpu.SMEM),
)


@jax.jit
def kernel(target_nodes, context_nodes_pos, context_nodes_neg,
           target_table, context_table):
    tgt = target_nodes.astype(jnp.int32).reshape(_NW, _TGT_CHUNKS, _CH)
    cp = context_nodes_pos.astype(jnp.int32).reshape(_NW, _CTX_CHUNKS, _CH)
    cn = context_nodes_neg.astype(jnp.int32).reshape(_NW, _CTX_CHUNKS, _CH)
    seg = (jnp.arange(_NS * _BPW * _L, dtype=jnp.int32) // _L).reshape(
        _NS, _CTX_CHUNKS, _CH)
    s_p, s_n = _sc_scores(tgt, cp, cn, seg, target_table, context_table)
    return _loss(s_p, s_n)[0, 0]
